# Initial kernel scaffold; baseline (speedup 1.0000x reference)
import jax
import jax.numpy as jnp
from jax.experimental import pallas as pl


def kernel(pos, W1a, b1a, W1b, b1b, W2a, b2a, W2b, b2b, W3a, b3a, W3b, b3b, Wh1, bh1, Wh2, bh2, Wh3, bh3):
    raise NotImplementedError("write your pallas kernel here")


# speedup vs baseline: 6.2743x; 6.2743x over previous
import functools

import jax
import jax.numpy as jnp
from jax import lax
from jax.experimental import pallas as pl
from jax.experimental.pallas import tpu as pltpu
from jax.experimental.pallas import tpu_sc as plsc

N = 10000
K = 16
F = 256

BQ = 200


def _knn_body(posq_ref, post_ref, out_ref):
    q = posq_ref[...]
    pt = post_ref[...]
    sq = jnp.sum(pt * pt, axis=0, keepdims=True)
    qq = jnp.sum(q * q, axis=1, keepdims=True)
    d2 = qq - 2.0 * lax.dot_general(q, pt, (((1,), (0,)), ((), ())),
                                    preferred_element_type=jnp.float32) + sq
    iota = lax.broadcasted_iota(jnp.int32, (BQ, N), 1)
    cols = []
    for _ in range(K):
        m = jnp.min(d2, axis=1, keepdims=True)
        idx = jnp.min(jnp.where(d2 == m, iota, N), axis=1)
        cols.append(idx[:, None])
        d2 = jnp.where(iota == idx[:, None], jnp.inf, d2)
    out_ref[...] = jnp.concatenate(cols, axis=1)


def _knn(pos, post):
    return pl.pallas_call(
        _knn_body,
        grid=(N // BQ,),
        in_specs=[
            pl.BlockSpec((BQ, 3), lambda i: (i, 0)),
            pl.BlockSpec((3, N), lambda i: (0, 0)),
        ],
        out_specs=pl.BlockSpec((BQ, K), lambda i: (i, 0)),
        out_shape=jax.ShapeDtypeStruct((N, K), jnp.int32),
    )(pos, post)


BT = 400


def _tables_body(pos_ref, w1_ref, w2p_ref, w3p_ref,
                 g1_ref, b1_ref, b2_ref, b3_ref):
    p = pos_ref[...]

    def mm(w):
        return lax.dot_general(p, w, (((1,), (0,)), ((), ())),
                               preferred_element_type=jnp.float32)

    w1 = w1_ref[...]
    b1 = mm(w1[3:6, :])
    g1_ref[...] = mm(w1[0:3, :]) + b1
    b1_ref[...] = b1
    b2_ref[...] = mm(w2p_ref[...])
    b3_ref[...] = mm(w3p_ref[...])


def _tables(pos, w1a, w2p, w3p):
    out = jax.ShapeDtypeStruct((N, F), jnp.float32)
    return pl.pallas_call(
        _tables_body,
        grid=(N // BT,),
        in_specs=[
            pl.BlockSpec((BT, 3), lambda i: (i, 0)),
            pl.BlockSpec((6, F), lambda i: (0, 0)),
            pl.BlockSpec((3, F), lambda i: (0, 0)),
            pl.BlockSpec((3, F), lambda i: (0, 0)),
        ],
        out_specs=[pl.BlockSpec((BT, F), lambda i: (i, 0))] * 4,
        out_shape=[out, out, out, out],
    )(pos, w1a, w2p, w3p)


_NC = 2
_NS = 16
_NW = _NC * _NS
_EDGES = N * K
_PER_W = _EDGES // _NW
_CH = 200
_NCH = _PER_W // _CH


def _sc_gather(table, idx):
    mesh = plsc.VectorSubcoreMesh(core_axis_name="c", subcore_axis_name="s")

    @functools.partial(
        pl.kernel,
        mesh=mesh,
        out_type=jax.ShapeDtypeStruct((_EDGES, F), jnp.float32),
        scratch_types=[
            pltpu.VMEM((_CH,), jnp.int32),
            pltpu.VMEM((_CH, F), jnp.float32),
            pltpu.SemaphoreType.DMA,
        ],
    )
    def gk(idx_hbm, table_hbm, out_hbm, idx_v, rows_v, sem):
        wid = lax.axis_index("s") * _NC + lax.axis_index("c")
        base = wid * _PER_W

        def body(c, carry):
            off = base + c * _CH
            pltpu.sync_copy(idx_hbm.at[pl.ds(off, _CH)], idx_v)
            pltpu.async_copy(table_hbm.at[idx_v], rows_v, sem).wait()
            pltpu.sync_copy(rows_v, out_hbm.at[pl.ds(off, _CH)])
            return carry

        lax.fori_loop(0, _NCH, body, 0)

    return gk(idx, table)


BN = 400


def _mmf(a, w):
    return lax.dot_general(a, w, (((1,), (0,)), ((), ())),
                           preferred_element_type=jnp.float32)


def _conv_core(x_ref, bdst_ref, ba_ref, wb_ref, bb_ref):
    bdst = bdst_ref[...]
    ba = ba_ref[...]
    wb = wb_ref[...]
    acc = jnp.full((BN, F), -jnp.inf, jnp.float32)
    for j in range(K):
        z = jnp.maximum(x_ref[j] - bdst + ba, 0.0)
        acc = jnp.maximum(acc, _mmf(z, wb))
    return jnp.maximum(acc + bb_ref[...], 0.0)


def _conv_g_body(x_ref, bdst_ref, ba_ref, wb_ref, bb_ref,
                 wnext_ref, bnext_ref, g_ref):
    h = _conv_core(x_ref, bdst_ref, ba_ref, wb_ref, bb_ref)
    g_ref[...] = _mmf(h, wnext_ref[...]) + bnext_ref[...]


def _conv_g(x, bdst, ba, wb, bb, wnext, bnext):
    return pl.pallas_call(
        _conv_g_body,
        grid=(N // BN,),
        in_specs=[
            pl.BlockSpec((K, BN, F), lambda i: (0, i, 0)),
            pl.BlockSpec((BN, F), lambda i: (i, 0)),
            pl.BlockSpec((1, F), lambda i: (0, 0)),
            pl.BlockSpec((F, F), lambda i: (0, 0)),
            pl.BlockSpec((1, F), lambda i: (0, 0)),
            pl.BlockSpec((F, F), lambda i: (0, 0)),
            pl.BlockSpec((BN, F), lambda i: (i, 0)),
        ],
        out_specs=pl.BlockSpec((BN, F), lambda i: (i, 0)),
        out_shape=jax.ShapeDtypeStruct((N, F), jnp.float32),
    )(x, bdst, ba, wb, bb, wnext, bnext)


def _conv_head_body(x_ref, bdst_ref, ba_ref, wb_ref, bb_ref,
                    wh1_ref, bh1_ref, wh2_ref, bh2_ref, wh3_ref, bh3_ref,
                    out_ref):
    h = _conv_core(x_ref, bdst_ref, ba_ref, wb_ref, bb_ref)
    t = jnp.maximum(_mmf(h, wh1_ref[...]) + bh1_ref[...], 0.0)
    t = jnp.maximum(_mmf(t, wh2_ref[...]) + bh2_ref[...], 0.0)
    o = _mmf(t, wh3_ref[...]) + bh3_ref[...]
    out_ref[...] = 1.0 / (1.0 + jnp.exp(-o))


def _conv_head(x, bdst, ba, wb, bb, wh1, bh1, wh2, bh2, wh3, bh3):
    return pl.pallas_call(
        _conv_head_body,
        grid=(N // BN,),
        in_specs=[
            pl.BlockSpec((K, BN, F), lambda i: (0, i, 0)),
            pl.BlockSpec((BN, F), lambda i: (i, 0)),
            pl.BlockSpec((1, F), lambda i: (0, 0)),
            pl.BlockSpec((F, F), lambda i: (0, 0)),
            pl.BlockSpec((1, F), lambda i: (0, 0)),
            pl.BlockSpec((F, 128), lambda i: (0, 0)),
            pl.BlockSpec((1, 128), lambda i: (0, 0)),
            pl.BlockSpec((128, 128), lambda i: (0, 0)),
            pl.BlockSpec((1, 128), lambda i: (0, 0)),
            pl.BlockSpec((128, 1), lambda i: (0, 0)),
            pl.BlockSpec((1, 1), lambda i: (0, 0)),
        ],
        out_specs=pl.BlockSpec((BN, 1), lambda i: (i, 0)),
        out_shape=jax.ShapeDtypeStruct((N, 1), jnp.float32),
    )(x, bdst, ba, wb, bb, wh1, bh1, wh2, bh2, wh3, bh3)


def kernel(pos, W1a, b1a, W1b, b1b, W2a, b2a, W2b, b2b, W3a, b3a, W3b, b3b,
           Wh1, bh1, Wh2, bh2, Wh3, bh3):
    post = pos.T
    nbr = _knn(pos, post)
    idx = nbr.T.reshape(-1)

    g1, b1t, b2t, b3t = _tables(pos, W1a, W2a[256:, :], W3a[256:, :])

    r = lambda b: b.reshape(1, -1)
    x = _sc_gather(g1, idx).reshape(K, N, F)
    g2 = _conv_g(x, b1t, r(b1a), W1b, r(b1b), W2a[:256, :], b2t)
    x = _sc_gather(g2, idx).reshape(K, N, F)
    g3 = _conv_g(x, b2t, r(b2a), W2b, r(b2b), W3a[:256, :], b3t)
    x = _sc_gather(g3, idx).reshape(K, N, F)
    return _conv_head(x, b3t, r(b3a), W3b, r(b3b),
                      Wh1, r(bh1), Wh2, r(bh2), Wh3, r(bh3))


# speedup vs baseline: 8.2850x; 1.3205x over previous
import functools

import jax
import jax.numpy as jnp
from jax import lax
from jax.experimental import pallas as pl
from jax.experimental.pallas import tpu as pltpu
from jax.experimental.pallas import tpu_sc as plsc

N = 10000
K = 16
F = 256

BQ = 200


def _knn_body(posq_ref, post_ref, out_ref):
    q = posq_ref[...]
    pt = post_ref[...]
    sq = jnp.sum(pt * pt, axis=0, keepdims=True)
    qq = jnp.sum(q * q, axis=1, keepdims=True)
    d2 = qq - 2.0 * lax.dot_general(q, pt, (((1,), (0,)), ((), ())),
                                    preferred_element_type=jnp.float32) + sq
    iota = lax.broadcasted_iota(jnp.int32, (BQ, N), 1)
    bits = lax.bitcast_convert_type(jnp.maximum(d2, 0.0), jnp.int32)
    key = (bits & jnp.int32(-16384)) | iota
    cols = []
    for _ in range(K):
        m = jnp.min(key, axis=1, keepdims=True)
        cols.append(m & jnp.int32(16383))
        key = jnp.where(key == m, jnp.int32(0x7FFFFFFF), key)
    out_ref[...] = jnp.concatenate(cols, axis=1)


def _knn(pos, post):
    return pl.pallas_call(
        _knn_body,
        grid=(N // BQ,),
        in_specs=[
            pl.BlockSpec((BQ, 3), lambda i: (i, 0)),
            pl.BlockSpec((3, N), lambda i: (0, 0)),
        ],
        out_specs=pl.BlockSpec((BQ, K), lambda i: (i, 0)),
        out_shape=jax.ShapeDtypeStruct((N, K), jnp.int32),
    )(pos, post)


BT = 400


def _tables_body(pos_ref, w1_ref, w2p_ref, w3p_ref,
                 g1_ref, b1_ref, b2_ref, b3_ref):
    p = pos_ref[...]

    def mm(w):
        return lax.dot_general(p, w, (((1,), (0,)), ((), ())),
                               preferred_element_type=jnp.float32)

    w1 = w1_ref[...]
    b1 = mm(w1[3:6, :])
    g1_ref[...] = mm(w1[0:3, :]) + b1
    b1_ref[...] = b1
    b2_ref[...] = mm(w2p_ref[...])
    b3_ref[...] = mm(w3p_ref[...])


def _tables(pos, w1a, w2p, w3p):
    out = jax.ShapeDtypeStruct((N, F), jnp.float32)
    return pl.pallas_call(
        _tables_body,
        grid=(N // BT,),
        in_specs=[
            pl.BlockSpec((BT, 3), lambda i: (i, 0)),
            pl.BlockSpec((6, F), lambda i: (0, 0)),
            pl.BlockSpec((3, F), lambda i: (0, 0)),
            pl.BlockSpec((3, F), lambda i: (0, 0)),
        ],
        out_specs=[pl.BlockSpec((BT, F), lambda i: (i, 0))] * 4,
        out_shape=[out, out, out, out],
    )(pos, w1a, w2p, w3p)


_NC = 2
_NS = 16
_NW = _NC * _NS
_EDGES = N * K
_PER_W = _EDGES // _NW
_CH = 200
_NCH = _PER_W // _CH


def _sc_gather(table, idx):
    mesh = plsc.VectorSubcoreMesh(core_axis_name="c", subcore_axis_name="s")

    @functools.partial(
        pl.kernel,
        mesh=mesh,
        out_type=jax.ShapeDtypeStruct((_EDGES, F), jnp.float32),
        scratch_types=[
            pltpu.VMEM((_CH,), jnp.int32),
            pltpu.VMEM((_CH, F), jnp.float32),
            pltpu.SemaphoreType.DMA,
        ],
    )
    def gk(idx_hbm, table_hbm, out_hbm, idx_v, rows_v, sem):
        wid = lax.axis_index("s") * _NC + lax.axis_index("c")
        base = wid * _PER_W

        def body(c, carry):
            off = base + c * _CH
            pltpu.sync_copy(idx_hbm.at[pl.ds(off, _CH)], idx_v)
            pltpu.async_copy(table_hbm.at[idx_v], rows_v, sem).wait()
            pltpu.sync_copy(rows_v, out_hbm.at[pl.ds(off, _CH)])
            return carry

        lax.fori_loop(0, _NCH, body, 0)

    return gk(idx, table)


BN = 400


def _mmf(a, w):
    return lax.dot_general(a, w, (((1,), (0,)), ((), ())),
                           preferred_element_type=jnp.float32)


def _conv_core(x_ref, bdst_ref, ba_ref, wb_ref, bb_ref):
    bdst = bdst_ref[...]
    ba = ba_ref[...]
    wb = wb_ref[...]
    acc = jnp.full((BN, F), -jnp.inf, jnp.float32)
    for j in range(K):
        z = jnp.maximum(x_ref[j] - bdst + ba, 0.0)
        acc = jnp.maximum(acc, _mmf(z, wb))
    return jnp.maximum(acc + bb_ref[...], 0.0)


def _conv_g_body(x_ref, bdst_ref, ba_ref, wb_ref, bb_ref,
                 wnext_ref, bnext_ref, g_ref):
    h = _conv_core(x_ref, bdst_ref, ba_ref, wb_ref, bb_ref)
    g_ref[...] = _mmf(h, wnext_ref[...]) + bnext_ref[...]


def _conv_g(x, bdst, ba, wb, bb, wnext, bnext):
    return pl.pallas_call(
        _conv_g_body,
        grid=(N // BN,),
        in_specs=[
            pl.BlockSpec((K, BN, F), lambda i: (0, i, 0)),
            pl.BlockSpec((BN, F), lambda i: (i, 0)),
            pl.BlockSpec((1, F), lambda i: (0, 0)),
            pl.BlockSpec((F, F), lambda i: (0, 0)),
            pl.BlockSpec((1, F), lambda i: (0, 0)),
            pl.BlockSpec((F, F), lambda i: (0, 0)),
            pl.BlockSpec((BN, F), lambda i: (i, 0)),
        ],
        out_specs=pl.BlockSpec((BN, F), lambda i: (i, 0)),
        out_shape=jax.ShapeDtypeStruct((N, F), jnp.float32),
    )(x, bdst, ba, wb, bb, wnext, bnext)


def _conv_head_body(x_ref, bdst_ref, ba_ref, wb_ref, bb_ref,
                    wh1_ref, bh1_ref, wh2_ref, bh2_ref, wh3_ref, bh3_ref,
                    out_ref):
    h = _conv_core(x_ref, bdst_ref, ba_ref, wb_ref, bb_ref)
    t = jnp.maximum(_mmf(h, wh1_ref[...]) + bh1_ref[...], 0.0)
    t = jnp.maximum(_mmf(t, wh2_ref[...]) + bh2_ref[...], 0.0)
    o = _mmf(t, wh3_ref[...]) + bh3_ref[...]
    out_ref[...] = 1.0 / (1.0 + jnp.exp(-o))


def _conv_head(x, bdst, ba, wb, bb, wh1, bh1, wh2, bh2, wh3, bh3):
    return pl.pallas_call(
        _conv_head_body,
        grid=(N // BN,),
        in_specs=[
            pl.BlockSpec((K, BN, F), lambda i: (0, i, 0)),
            pl.BlockSpec((BN, F), lambda i: (i, 0)),
            pl.BlockSpec((1, F), lambda i: (0, 0)),
            pl.BlockSpec((F, F), lambda i: (0, 0)),
            pl.BlockSpec((1, F), lambda i: (0, 0)),
            pl.BlockSpec((F, 128), lambda i: (0, 0)),
            pl.BlockSpec((1, 128), lambda i: (0, 0)),
            pl.BlockSpec((128, 128), lambda i: (0, 0)),
            pl.BlockSpec((1, 128), lambda i: (0, 0)),
            pl.BlockSpec((128, 1), lambda i: (0, 0)),
            pl.BlockSpec((1, 1), lambda i: (0, 0)),
        ],
        out_specs=pl.BlockSpec((BN, 1), lambda i: (i, 0)),
        out_shape=jax.ShapeDtypeStruct((N, 1), jnp.float32),
    )(x, bdst, ba, wb, bb, wh1, bh1, wh2, bh2, wh3, bh3)


def kernel(pos, W1a, b1a, W1b, b1b, W2a, b2a, W2b, b2b, W3a, b3a, W3b, b3b,
           Wh1, bh1, Wh2, bh2, Wh3, bh3):
    post = pos.T
    nbr = _knn(pos, post)
    idx = nbr.T.reshape(-1)

    g1, b1t, b2t, b3t = _tables(pos, W1a, W2a[256:, :], W3a[256:, :])

    r = lambda b: b.reshape(1, -1)
    x = _sc_gather(g1, idx).reshape(K, N, F)
    g2 = _conv_g(x, b1t, r(b1a), W1b, r(b1b), W2a[:256, :], b2t)
    x = _sc_gather(g2, idx).reshape(K, N, F)
    g3 = _conv_g(x, b2t, r(b2a), W2b, r(b2b), W3a[:256, :], b3t)
    x = _sc_gather(g3, idx).reshape(K, N, F)
    return _conv_head(x, b3t, r(b3a), W3b, r(b3b),
                      Wh1, r(bh1), Wh2, r(bh2), Wh3, r(bh3))
